# Initial kernel scaffold; baseline (speedup 1.0000x reference)
import jax
import jax.numpy as jnp
from jax.experimental import pallas as pl


def kernel(head, relation, tail, head_w, rel_w, tail_w, edge_index, entity_emb, relation_emb, word_emb, e_bias, r_bias, W1, b1, W2, b2, W3, b3):
    raise NotImplementedError("write your pallas kernel here")


# speedup vs baseline: 5.9042x; 5.9042x over previous
import functools

import jax
import jax.numpy as jnp
from jax import lax
from jax.experimental import pallas as pl
from jax.experimental.pallas import tpu as pltpu
from jax.experimental.pallas import tpu_sc as plsc

N = 10000
NPAD = 10240
E = 160000
D = 256
H = 128

_MESH = plsc.VectorSubcoreMesh(core_axis_name="c", subcore_axis_name="s")


def _embed_body(head_p, rel_p, tail_p, hw, rw, tw, dst_e, ent, rel_t, wrd8,
                ebias, rbias, onesh,
                x0, rele, bsum, degp,
                dacc, idxv, wjv, xout, bacc, didx, onesv, stg, sem):
    c = lax.axis_index("c")
    s = lax.axis_index("s")
    wid = c * 16 + s

    def zdeg(i, carry):
        stg[pl.ds(i * 16, 16)] = jnp.zeros((16,), jnp.float32)
        return carry

    lax.fori_loop(0, 40, zdeg, 0)
    pltpu.sync_copy(stg, dacc.at[pl.ds(s * 640, 640)])
    pltpu.sync_copy(onesh, onesv)
    plsc.subcore_barrier()

    ebase = c * 80000 + s * 5000

    def deg_win(w, carry):
        pltpu.sync_copy(dst_e.at[pl.ds(ebase + w * 1000, 1000)], didx)
        pltpu.sync_copy(onesv, dacc.at[didx], add=True)
        return carry

    lax.fori_loop(0, 5, deg_win, 0)
    plsc.subcore_barrier()
    pltpu.sync_copy(dacc.at[pl.ds(s * 640, 640)], stg)
    pltpu.sync_copy(stg, degp.at[pl.ds(c * NPAD + s * 640, 640)])

    sb = wid * 320

    pltpu.sync_copy(head_p.at[pl.ds(sb, 320)], idxv)
    pltpu.async_copy(ent.at[idxv], xout, sem).wait()
    pltpu.async_copy(ebias.at[idxv], bacc, sem).wait()
    for j in range(8):
        pltpu.sync_copy(hw.at[pl.ds(j * NPAD + sb, 320)], wjv)
        pltpu.sync_copy(wrd8.at[wjv], xout, add=True)
    pltpu.sync_copy(xout, x0.at[0, pl.ds(sb, 320), :])

    pltpu.sync_copy(tail_p.at[pl.ds(sb, 320)], idxv)
    pltpu.async_copy(ent.at[idxv], xout, sem).wait()
    pltpu.sync_copy(ebias.at[idxv], bacc, add=True)
    for j in range(8):
        pltpu.sync_copy(tw.at[pl.ds(j * NPAD + sb, 320)], wjv)
        pltpu.sync_copy(wrd8.at[wjv], xout, add=True)
    pltpu.sync_copy(xout, x0.at[1, pl.ds(sb, 320), :])

    pltpu.sync_copy(rel_p.at[pl.ds(sb, 320)], idxv)
    pltpu.async_copy(rel_t.at[idxv], xout, sem).wait()
    pltpu.sync_copy(rbias.at[idxv], bacc, add=True)
    for j in range(8):
        pltpu.sync_copy(rw.at[pl.ds(j * NPAD + sb, 320)], wjv)
        pltpu.sync_copy(wrd8.at[wjv], xout, add=True)
    pltpu.sync_copy(xout, rele.at[pl.ds(sb, 320), :])

    pltpu.sync_copy(bacc, bsum.at[pl.ds(sb, 320)])


_embed_call = pl.kernel(
    _embed_body,
    out_type=(
        jax.ShapeDtypeStruct((2, NPAD, D), jnp.float32),
        jax.ShapeDtypeStruct((NPAD, D), jnp.float32),
        jax.ShapeDtypeStruct((NPAD,), jnp.float32),
        jax.ShapeDtypeStruct((2 * NPAD,), jnp.float32),
    ),
    mesh=_MESH,
    scratch_types=[
        pltpu.VMEM_SHARED((NPAD,), jnp.float32),
        pltpu.VMEM((320,), jnp.int32),
        pltpu.VMEM((320,), jnp.int32),
        pltpu.VMEM((320, D), jnp.float32),
        pltpu.VMEM((320,), jnp.float32),
        pltpu.VMEM((1000,), jnp.int32),
        pltpu.VMEM((1000,), jnp.float32),
        pltpu.VMEM((640,), jnp.float32),
        pltpu.SemaphoreType.DMA,
    ],
)


_ADD_STARTS = tuple(range(0, 192, 16)) + (184,)


def _conv_body(xv, src, dst, y,
               acc, sidxr, sidxa, didxr, rows, zstg, ostg, sem):
    c = lax.axis_index("c")
    s = lax.axis_index("s")

    def zrow(i, carry):
        zstg[i, :] = jnp.zeros((H,), jnp.float32)
        return carry

    lax.fori_loop(0, 64, zrow, 0)

    def chunk(k, carry):
        off = (c * 2 + k) * NPAD
        for i in range(10):
            pltpu.sync_copy(zstg, acc.at[pl.ds(s * 640 + i * 64, 64), :])
        plsc.subcore_barrier()

        ebase = s * 10000

        def win(w, carry2):
            e0 = ebase + w * 200
            pltpu.sync_copy(src.at[pl.ds(e0, 200)], sidxr)
            pltpu.sync_copy(dst.at[pl.ds(e0, 200)], didxr)
            for st in _ADD_STARTS:
                sidxa[pl.ds(st, 16)] = sidxr[pl.ds(st, 16)] + off
            pltpu.async_copy(xv.at[sidxa], rows, sem).wait()
            pltpu.sync_copy(rows, acc.at[didxr], add=True)
            return carry2

        lax.fori_loop(0, 50, win, 0)
        plsc.subcore_barrier()
        for i in range(10):
            r0 = s * 640 + i * 64
            pltpu.sync_copy(acc.at[pl.ds(r0, 64), :], ostg)
            pltpu.sync_copy(ostg, y.at[pl.ds(off + r0, 64), :])
        plsc.subcore_barrier()
        return carry

    lax.fori_loop(0, 2, chunk, 0)


_conv_call = pl.kernel(
    _conv_body,
    out_type=jax.ShapeDtypeStruct((4 * NPAD, H), jnp.float32),
    mesh=_MESH,
    scratch_types=[
        pltpu.VMEM_SHARED((NPAD, H), jnp.float32),
        pltpu.VMEM((200,), jnp.int32),
        pltpu.VMEM((200,), jnp.int32),
        pltpu.VMEM((200,), jnp.int32),
        pltpu.VMEM((200, H), jnp.float32),
        pltpu.VMEM((64, H), jnp.float32),
        pltpu.VMEM((64, H), jnp.float32),
        pltpu.SemaphoreType.DMA,
    ],
)


def _dinv_body(p0_ref, p1_ref, o_ref):
    deg = p0_ref[...] + p1_ref[...] + 1.0
    o_ref[...] = lax.rsqrt(deg)[:, None]


def _dinv(degp):
    return pl.pallas_call(
        _dinv_body,
        grid=(10,),
        in_specs=[
            pl.BlockSpec((1024,), lambda r: (r,)),
            pl.BlockSpec((1024,), lambda r: (r + 10,)),
        ],
        out_specs=pl.BlockSpec((1024, 1), lambda r: (r, 0)),
        out_shape=jax.ShapeDtypeStruct((NPAD, 1), jnp.float32),
    )(degp, degp)


def _matmul_body(x_ref, w_ref, dinv_ref, o_ref):
    xw = jnp.dot(x_ref[0], w_ref[...], preferred_element_type=jnp.float32)
    o_ref[...] = xw * dinv_ref[...]


def _matmul_scale(x, w, dinv):
    return pl.pallas_call(
        _matmul_body,
        grid=(2, 10, 2),
        in_specs=[
            pl.BlockSpec((1, 1024, D), lambda s, r, h: (s, r, 0)),
            pl.BlockSpec((D, H), lambda s, r, h: (0, h)),
            pl.BlockSpec((1024, 1), lambda s, r, h: (r, 0)),
        ],
        out_specs=pl.BlockSpec((1024, H), lambda s, r, h: (s * 20 + h * 10 + r, 0)),
        out_shape=jax.ShapeDtypeStruct((4 * NPAD, H), jnp.float32),
    )(x, w, dinv)


def _combine_body(act, y_ref, xv_ref, dinv_ref, b_ref, o_ref):
    v = dinv_ref[...] * (y_ref[...] + xv_ref[...]) + b_ref[...]
    if act:
        v = jnp.maximum(v, 0.0)
    o_ref[...] = v[None]


def _combine(y, xv, dinv, b, act):
    return pl.pallas_call(
        functools.partial(_combine_body, act),
        grid=(2, 10, 2),
        in_specs=[
            pl.BlockSpec((1024, H), lambda s, r, h: (s * 20 + h * 10 + r, 0)),
            pl.BlockSpec((1024, H), lambda s, r, h: (s * 20 + h * 10 + r, 0)),
            pl.BlockSpec((1024, 1), lambda s, r, h: (r, 0)),
            pl.BlockSpec((1, H), lambda s, r, h: (0, h)),
        ],
        out_specs=pl.BlockSpec((1, 1024, H), lambda s, r, h: (s, r, h)),
        out_shape=jax.ShapeDtypeStruct((2, NPAD, D), jnp.float32),
    )(y, xv, dinv, b.reshape(1, D))


def _score_body(x3_ref, rel_ref, bias_ref, o_ref):
    z = x3_ref[0] + rel_ref[...] - x3_ref[1]
    ss = jnp.sum(z * z, axis=1, keepdims=True)
    o_ref[...] = bias_ref[...] - jnp.sqrt(ss)


def _score(x3, rel_e, bias):
    return pl.pallas_call(
        _score_body,
        grid=(10,),
        in_specs=[
            pl.BlockSpec((2, 1000, D), lambda r: (0, r, 0)),
            pl.BlockSpec((1000, D), lambda r: (r, 0)),
            pl.BlockSpec((1000, 1), lambda r: (r, 0)),
        ],
        out_specs=pl.BlockSpec((1000, 1), lambda r: (r, 0)),
        out_shape=jax.ShapeDtypeStruct((N, 1), jnp.float32),
    )(x3, rel_e, bias)


def kernel(head, relation, tail, head_w, rel_w, tail_w, edge_index,
           entity_emb, relation_emb, word_emb, e_bias, r_bias,
           W1, b1, W2, b2, W3, b3):
    i32 = jnp.int32
    pad = NPAD - N
    head_p = jnp.pad(head.astype(i32), (0, pad))
    rel_p = jnp.pad(relation.astype(i32), (0, pad))
    tail_p = jnp.pad(tail.astype(i32), (0, pad))
    hw = jnp.pad(head_w.astype(i32), ((0, pad), (0, 0))).T.reshape(-1)
    rw = jnp.pad(rel_w.astype(i32), ((0, pad), (0, 0))).T.reshape(-1)
    tw = jnp.pad(tail_w.astype(i32), ((0, pad), (0, 0))).T.reshape(-1)
    src = edge_index[0].astype(i32)
    dst = edge_index[1].astype(i32)

    wrd8 = word_emb * 0.125

    onesh = jnp.ones((1000,), jnp.float32)

    he = entity_emb[head] + jnp.mean(word_emb[head_w], axis=1)
    te = entity_emb[tail] + jnp.mean(word_emb[tail_w], axis=1)
    re_ = relation_emb[relation] + jnp.mean(word_emb[rel_w], axis=1)
    x = jnp.zeros((2, NPAD, D), jnp.float32).at[0, :N].set(he).at[1, :N].set(te)
    rel_e = jnp.zeros((NPAD, D), jnp.float32).at[:N].set(re_)
    bias = jnp.zeros((NPAD, 1), jnp.float32).at[:N, 0].set(
        e_bias[head][:, 0] + e_bias[tail][:, 0] + r_bias[relation][:, 0])
    deg = jnp.zeros((NPAD,), jnp.float32).at[dst].add(1.0)
    dinv = lax.rsqrt(deg + 1.0).reshape(NPAD, 1)

    for w, b, act in ((W1, b1, True), (W2, b2, True), (W3, b3, False)):
        xv = _matmul_scale(x, w, dinv)
        y = _conv_call(xv, src, dst)
        x = _combine(y, xv, dinv, b, act)

    return _score(x, rel_e, bias)[:, 0]
